# Initial kernel scaffold; baseline (speedup 1.0000x reference)
import jax
import jax.numpy as jnp
from jax.experimental import pallas as pl


def kernel(P, d_error, circuit_edge_pairs, circuit_edge_weights):
    raise NotImplementedError("write your pallas kernel here")


# speedup vs baseline: 25.5759x; 25.5759x over previous
import functools

import jax
import jax.numpy as jnp
from jax import lax
from jax.experimental import pallas as pl
from jax.experimental.pallas import tpu as pltpu
from jax.experimental.pallas import tpu_sc as plsc

B, E, N = 64, 1024, 128
NC, NS, L = 2, 16, 16
NW = NC * NS
BPW = B // NW


def _tc_q_body(p_ref, d_ref, q_ref):
    p = p_ref[0]
    m = jnp.dot(p, d_ref[...], preferred_element_type=jnp.float32)
    q_ref[0] = lax.dot_general(m, p, (((1,), (1,)), ((), ())),
                               preferred_element_type=jnp.float32)


def _compute_q(P, d_error):
    return pl.pallas_call(
        _tc_q_body,
        grid=(B,),
        in_specs=[
            pl.BlockSpec((1, N, N), lambda b: (b, 0, 0)),
            pl.BlockSpec((N, N), lambda b: (0, 0)),
        ],
        out_specs=pl.BlockSpec((1, N, N), lambda b: (b, 0, 0)),
        out_shape=jax.ShapeDtypeStruct((B, N, N), jnp.float32),
    )(P, d_error)


@functools.partial(
    pl.kernel,
    out_type=jax.ShapeDtypeStruct((B, L), jnp.float32),
    mesh=plsc.VectorSubcoreMesh(core_axis_name="c", subcore_axis_name="s",
                                num_cores=NC, num_subcores=NS),
    compiler_params=pltpu.CompilerParams(needs_layout_passes=False),
    scratch_types=[
        pltpu.VMEM((N, N), jnp.float32),
        pltpu.VMEM((2 * E,), jnp.int32),
        pltpu.VMEM((E,), jnp.float32),
        pltpu.VMEM((L,), jnp.float32),
    ],
)
def _sc_edge_reduce(q_hbm, pairs_hbm, w_hbm, out_hbm, q_v, pairs_v, w_v, out_v):
    wid = lax.axis_index("s") * NC + lax.axis_index("c")
    lanes = lax.iota(jnp.int32, L)
    for local in range(BPW):
        b = wid * BPW + local
        pltpu.sync_copy(q_hbm.at[b], q_v)
        pltpu.sync_copy(pairs_hbm.at[b], pairs_v)
        pltpu.sync_copy(w_hbm.at[b], w_v)

        def body(k, carry):
            acc, wsum = carry
            ev = lanes * 2 + k * (2 * L)
            i_vec = plsc.load_gather(pairs_v, [ev])
            j_vec = plsc.load_gather(pairs_v, [ev + 1])
            vals = plsc.load_gather(q_v, [i_vec, j_vec])
            wk = plsc.load_gather(w_v, [lanes + k * L])
            return acc + wk * vals, wsum + wk

        acc, wsum = lax.fori_loop(
            0, E // L, body,
            (jnp.zeros((L,), jnp.float32), jnp.zeros((L,), jnp.float32)))
        svec = jnp.full((L,), jnp.sum(acc), jnp.float32)
        wvec = jnp.full((L,), jnp.maximum(jnp.sum(wsum), 1e-8), jnp.float32)
        out_v[...] = svec / wvec
        pltpu.sync_copy(out_v, out_hbm.at[b])


def kernel(P, d_error, circuit_edge_pairs, circuit_edge_weights):
    Q = _compute_q(P, d_error)
    pairs_flat = circuit_edge_pairs.reshape(B, 2 * E)
    per_sample = _sc_edge_reduce(Q, pairs_flat, circuit_edge_weights)
    return jnp.sum(per_sample[:, 0]) / B


# speedup vs baseline: 45.9174x; 1.7953x over previous
import functools

import jax
import jax.numpy as jnp
from jax import lax
from jax.experimental import pallas as pl
from jax.experimental.pallas import tpu as pltpu
from jax.experimental.pallas import tpu_sc as plsc

B, E, N = 64, 1024, 128
NC, NS, L = 2, 16, 16
NW = NC * NS
BPW = B // NW


G = 8


def _tc_q_body(p_ref, d_ref, q_ref):
    d = d_ref[...]
    for g in range(G):
        p = p_ref[g]
        m = jnp.dot(p, d, preferred_element_type=jnp.float32)
        q_ref[g] = lax.dot_general(m, p, (((1,), (1,)), ((), ())),
                                   preferred_element_type=jnp.float32)


def _compute_q(P, d_error):
    return pl.pallas_call(
        _tc_q_body,
        grid=(B // G,),
        in_specs=[
            pl.BlockSpec((G, N, N), lambda b: (b, 0, 0)),
            pl.BlockSpec((N, N), lambda b: (0, 0)),
        ],
        out_specs=pl.BlockSpec((G, N, N), lambda b: (b, 0, 0)),
        out_shape=jax.ShapeDtypeStruct((B, N, N), jnp.float32),
    )(P, d_error)


@functools.partial(
    pl.kernel,
    out_type=jax.ShapeDtypeStruct((B, L), jnp.float32),
    mesh=plsc.VectorSubcoreMesh(core_axis_name="c", subcore_axis_name="s",
                                num_cores=NC, num_subcores=NS),
    compiler_params=pltpu.CompilerParams(needs_layout_passes=False),
    scratch_types=[
        pltpu.VMEM((N, N), jnp.float32),
        pltpu.VMEM((2 * E,), jnp.int32),
        pltpu.VMEM((E,), jnp.float32),
        pltpu.VMEM((L,), jnp.float32),
    ],
)
def _sc_edge_reduce(q_hbm, pairs_hbm, w_hbm, out_hbm, q_v, pairs_v, w_v, out_v):
    wid = lax.axis_index("s") * NC + lax.axis_index("c")
    lanes = lax.iota(jnp.int32, L)
    for local in range(BPW):
        b = wid * BPW + local
        pltpu.sync_copy(q_hbm.at[b], q_v)
        pltpu.sync_copy(pairs_hbm.at[b], pairs_v)
        pltpu.sync_copy(w_hbm.at[b], w_v)

        def body(k, carry):
            acc, wsum = carry
            ev = lanes * 2 + k * (2 * L)
            i_vec = plsc.load_gather(pairs_v, [ev])
            j_vec = plsc.load_gather(pairs_v, [ev + 1])
            vals = plsc.load_gather(q_v, [i_vec, j_vec])
            wk = plsc.load_gather(w_v, [lanes + k * L])
            return acc + wk * vals, wsum + wk

        acc, wsum = lax.fori_loop(
            0, E // L, body,
            (jnp.zeros((L,), jnp.float32), jnp.zeros((L,), jnp.float32)))
        svec = jnp.full((L,), jnp.sum(acc), jnp.float32)
        wvec = jnp.full((L,), jnp.maximum(jnp.sum(wsum), 1e-8), jnp.float32)
        out_v[...] = svec / wvec
        pltpu.sync_copy(out_v, out_hbm.at[b])


def kernel(P, d_error, circuit_edge_pairs, circuit_edge_weights):
    Q = _compute_q(P, d_error)
    pairs_flat = circuit_edge_pairs.reshape(B, 2 * E)
    per_sample = _sc_edge_reduce(Q, pairs_flat, circuit_edge_weights)
    return jnp.sum(per_sample[:, 0]) / B


# speedup vs baseline: 47.7480x; 1.0399x over previous
import functools

import jax
import jax.numpy as jnp
from jax import lax
from jax.experimental import pallas as pl
from jax.experimental.pallas import tpu as pltpu
from jax.experimental.pallas import tpu_sc as plsc

B, E, N = 64, 1024, 128
NC, NS, L = 2, 16, 16
NW = NC * NS
BPW = B // NW


G = 16


def _tc_q_body(p_ref, d_ref, q_ref):
    d = d_ref[...]
    for g in range(G):
        p = p_ref[g]
        m = jnp.dot(p, d, preferred_element_type=jnp.float32)
        q_ref[g] = lax.dot_general(m, p, (((1,), (1,)), ((), ())),
                                   preferred_element_type=jnp.float32)


def _compute_q(P, d_error):
    return pl.pallas_call(
        _tc_q_body,
        grid=(B // G,),
        in_specs=[
            pl.BlockSpec((G, N, N), lambda b: (b, 0, 0)),
            pl.BlockSpec((N, N), lambda b: (0, 0)),
        ],
        out_specs=pl.BlockSpec((G, N, N), lambda b: (b, 0, 0)),
        out_shape=jax.ShapeDtypeStruct((B, N, N), jnp.float32),
    )(P, d_error)


@functools.partial(
    pl.kernel,
    out_type=jax.ShapeDtypeStruct((B, L), jnp.float32),
    mesh=plsc.VectorSubcoreMesh(core_axis_name="c", subcore_axis_name="s",
                                num_cores=NC, num_subcores=NS),
    compiler_params=pltpu.CompilerParams(needs_layout_passes=False),
    scratch_types=[
        pltpu.VMEM((N, N), jnp.float32),
        pltpu.VMEM((2 * E,), jnp.int32),
        pltpu.VMEM((E,), jnp.float32),
        pltpu.VMEM((L,), jnp.float32),
    ],
)
def _sc_edge_reduce(q_hbm, pairs_hbm, w_hbm, out_hbm, q_v, pairs_v, w_v, out_v):
    wid = lax.axis_index("s") * NC + lax.axis_index("c")
    lanes = lax.iota(jnp.int32, L)
    for local in range(BPW):
        b = wid * BPW + local
        pltpu.sync_copy(q_hbm.at[b], q_v)
        pltpu.sync_copy(pairs_hbm.at[b], pairs_v)
        pltpu.sync_copy(w_hbm.at[b], w_v)

        def body(k, carry):
            acc, wsum = carry
            ev = lanes * 2 + k * (2 * L)
            i_vec = plsc.load_gather(pairs_v, [ev])
            j_vec = plsc.load_gather(pairs_v, [ev + 1])
            vals = plsc.load_gather(q_v, [i_vec, j_vec])
            wk = plsc.load_gather(w_v, [lanes + k * L])
            return acc + wk * vals, wsum + wk

        acc, wsum = lax.fori_loop(
            0, E // L, body,
            (jnp.zeros((L,), jnp.float32), jnp.zeros((L,), jnp.float32)))
        svec = jnp.full((L,), jnp.sum(acc), jnp.float32)
        wvec = jnp.full((L,), jnp.maximum(jnp.sum(wsum), 1e-8), jnp.float32)
        out_v[...] = svec / wvec
        pltpu.sync_copy(out_v, out_hbm.at[b])


def kernel(P, d_error, circuit_edge_pairs, circuit_edge_weights):
    Q = _compute_q(P, d_error)
    pairs_flat = circuit_edge_pairs.reshape(B, 2 * E)
    per_sample = _sc_edge_reduce(Q, pairs_flat, circuit_edge_weights)
    return jnp.sum(per_sample[:, 0]) / B
